# Initial kernel scaffold; baseline (speedup 1.0000x reference)
import jax
import jax.numpy as jnp
from jax.experimental import pallas as pl


def kernel(x_embed, prompt, prompt_key, conv_w, conv_b):
    raise NotImplementedError("write your pallas kernel here")


# speedup vs baseline: 2.5288x; 2.5288x over previous
import functools

import jax
import jax.numpy as jnp
from jax import lax
from jax.experimental import pallas as pl
from jax.experimental.pallas import tpu as pltpu
from jax.experimental.pallas import tpu_sc as plsc

B, L, D = 256, 196, 768
POOL, C, SIZE, PATCH, K = 8192, 3, 32, 16, 8
NP = (SIZE // PATCH) ** 2
PLEN = K * NP
ROW = C * SIZE * SIZE
SEL = B * K
OUT_L = PLEN + L

BB = 8
PN = 2048
PB = POOL // PN


def _mean_copy_body(x_ref, out_ref, xn_ref):
    x = x_ref[...]
    out_ref[:, PLEN:, :] = x
    m = jnp.sum(x, axis=1) * (1.0 / L)
    ss = jnp.sum(m * m, axis=1, keepdims=True)
    xn_ref[...] = m * lax.rsqrt(jnp.maximum(ss, 1e-12))


def _mean_copy(x_embed):
    return pl.pallas_call(
        _mean_copy_body,
        grid=(B // BB,),
        in_specs=[pl.BlockSpec((BB, L, D), lambda i: (i, 0, 0))],
        out_specs=[
            pl.BlockSpec((BB, OUT_L, D), lambda i: (i, 0, 0)),
            pl.BlockSpec((BB, D), lambda i: (i, 0)),
        ],
        out_shape=[
            jax.ShapeDtypeStruct((B, OUT_L, D), jnp.float32),
            jax.ShapeDtypeStruct((B, D), jnp.float32),
        ],
    )(x_embed)


def _sim_topk_body(xn_ref, key_ref, sim_ref, idx_ref, rsim_ref):
    j = pl.program_id(0)
    key = key_ref[...]
    kn = key * lax.rsqrt(
        jnp.maximum(jnp.sum(key * key, axis=1, keepdims=True), 1e-12))
    xn = xn_ref[...]
    s = lax.dot_general(xn, kn, (((1,), (1,)), ((), ())),
                        preferred_element_type=jnp.float32)
    sim_ref[:, pl.ds(j * PN, PN)] = s

    @pl.when(j == PB - 1)
    def _():
        sm = sim_ref[...]
        iota = lax.broadcasted_iota(jnp.int32, (B, POOL), 1)
        vals, idxs = [], []
        for _ in range(K):
            m = jnp.max(sm, axis=1, keepdims=True)
            am = jnp.min(jnp.where(sm == m, iota, POOL), axis=1,
                         keepdims=True)
            vals.append(m)
            idxs.append(am)
            sm = jnp.where(iota == am, -jnp.inf, sm)
        idx_ref[...] = jnp.concatenate(idxs, axis=1)
        rsim_ref[...] = (jnp.sum(jnp.concatenate(vals, axis=1))
                         * (1.0 / B)).reshape(1, 1)


def _sim_topk(xn, prompt_key):
    return pl.pallas_call(
        _sim_topk_body,
        grid=(PB,),
        in_specs=[
            pl.BlockSpec((B, D), lambda j: (0, 0)),
            pl.BlockSpec((PN, D), lambda j: (j, 0)),
        ],
        out_specs=[
            pl.BlockSpec((B, POOL), lambda j: (0, 0)),
            pl.BlockSpec((B, K), lambda j: (0, 0)),
            pl.BlockSpec((1, 1), lambda j: (0, 0)),
        ],
        out_shape=[
            jax.ShapeDtypeStruct((B, POOL), jnp.float32),
            jax.ShapeDtypeStruct((B, K), jnp.int32),
            jax.ShapeDtypeStruct((1, 1), jnp.float32),
        ],
    )(xn, prompt_key)


_NC, _NS = 2, 16
_NW = _NC * _NS
_BPW = SEL // _NW
_CH = 16
_NCH = _BPW // _CH


def _sc_gather_body(tbl_hbm, idx_hbm, out_hbm, idx_v, buf_v, sem0, sem1):
    wid = lax.axis_index("s") * _NC + lax.axis_index("c")
    base = wid * _BPW
    pltpu.sync_copy(idx_hbm.at[pl.ds(base, _BPW)], idx_v)
    sems = (sem0, sem1)
    copies = [None] * _NCH
    copies[0] = pltpu.async_copy(
        tbl_hbm.at[idx_v.at[pl.ds(0, _CH)]], buf_v.at[0], sems[0])
    for c in range(_NCH):
        if c + 1 < _NCH:
            copies[c + 1] = pltpu.async_copy(
                tbl_hbm.at[idx_v.at[pl.ds((c + 1) * _CH, _CH)]],
                buf_v.at[(c + 1) % 2], sems[(c + 1) % 2])
        copies[c].wait()
        pltpu.sync_copy(buf_v.at[c % 2], out_hbm.at[pl.ds(base + c * _CH, _CH)])


def _sc_gather(tbl, flat_idx):
    fn = functools.partial(
        pl.kernel,
        mesh=plsc.VectorSubcoreMesh(core_axis_name="c", subcore_axis_name="s"),
        out_type=jax.ShapeDtypeStruct((SEL, ROW), jnp.float32),
        scratch_types=[
            pltpu.VMEM((_BPW,), jnp.int32),
            pltpu.VMEM((2, _CH, ROW), jnp.float32),
            pltpu.SemaphoreType.DMA,
            pltpu.SemaphoreType.DMA,
        ],
    )(_sc_gather_body)
    return fn(tbl, flat_idx)


def _conv_body(p_ref, w_ref, b_ref, _pbuf_ref, out_ref):
    p = p_ref[...].reshape(BB * PLEN, C * PATCH * PATCH)
    y = lax.dot_general(p, w_ref[...], (((1,), (1,)), ((), ())),
                        preferred_element_type=jnp.float32)
    y = y + b_ref[...]
    out_ref[...] = y.reshape(BB, PLEN, D)


def _conv_assemble(patches, w2, b2, pbuf):
    return pl.pallas_call(
        _conv_body,
        grid=(B // BB,),
        in_specs=[
            pl.BlockSpec((BB, PLEN, D), lambda i: (i, 0, 0)),
            pl.BlockSpec((D, C * PATCH * PATCH), lambda i: (0, 0)),
            pl.BlockSpec((1, D), lambda i: (0, 0)),
            pl.BlockSpec((BB, PLEN, D), lambda i: (i, 0, 0)),
        ],
        out_specs=pl.BlockSpec((BB, PLEN, D), lambda i: (i, 0, 0)),
        out_shape=jax.ShapeDtypeStruct((B, OUT_L, D), jnp.float32),
        input_output_aliases={3: 0},
    )(patches, w2, b2, pbuf)


def kernel(x_embed, prompt, prompt_key, conv_w, conv_b):
    pbuf, xn = _mean_copy(x_embed)
    similarity, idx, rsim = _sim_topk(xn, prompt_key)
    flat_idx = idx.reshape(SEL)
    g = _sc_gather(prompt.reshape(POOL, ROW), flat_idx)
    prompt_image = g.reshape(B, K, C, SIZE, SIZE)
    patches = (
        g.reshape(SEL, C, SIZE // PATCH, PATCH, SIZE // PATCH, PATCH)
        .transpose(0, 2, 4, 1, 3, 5)
        .reshape(B, PLEN, C * PATCH * PATCH))
    w2 = conv_w.reshape(D, C * PATCH * PATCH)
    b2 = conv_b.reshape(1, D)
    prompted = _conv_assemble(patches, w2, b2, pbuf)
    return prompted, rsim[0, 0], similarity, idx, prompt_image
